# Initial kernel scaffold; baseline (speedup 1.0000x reference)
import jax
import jax.numpy as jnp
from jax.experimental import pallas as pl


def kernel(atom_fea, nbr_fea, nbr_fea_idx, crystal_atom_idx, atom_type, nbr_type, nbr_dist, pair_type, global_fea, params):
    raise NotImplementedError("write your pallas kernel here")


# speedup vs baseline: 1.4956x; 1.4956x over previous
import functools

import jax
import jax.numpy as jnp
from jax import lax
from jax.experimental import pallas as pl
from jax.experimental.pallas import tpu as pltpu
from jax.experimental.pallas import tpu_sc as plsc

F32 = jnp.float32
AFL = 64
M = 16
NBR = 16
PER = 100

_CH = 128


def _softplus(x):
    return jnp.maximum(x, 0.0) + jnp.log(1.0 + jnp.exp(-jnp.abs(x)))


def _sigmoid(x):
    z = jnp.exp(-jnp.abs(x))
    return jnp.where(x >= 0, 1.0 / (1.0 + z), z / (1.0 + z))


def _make_gather(n_rows, n_edges, feat):
    info = plsc.get_sparse_core_info()
    nw = info.num_cores * info.num_subcores
    n_chunks = n_edges // _CH
    assert n_chunks * _CH == n_edges and n_chunks % nw == 0
    cpw = n_chunks // nw
    mesh = plsc.VectorSubcoreMesh(core_axis_name="c", subcore_axis_name="s")

    @functools.partial(
        pl.kernel,
        mesh=mesh,
        compiler_params=pltpu.CompilerParams(use_tc_tiling_on_sc=False),
        out_type=jax.ShapeDtypeStruct((n_edges, feat), F32),
        scratch_types=[
            pltpu.VMEM((cpw, _CH), jnp.int32),
            pltpu.VMEM((2, _CH, feat), F32),
            pltpu.SemaphoreType.DMA,
            pltpu.SemaphoreType.DMA,
        ],
    )
    def gk(x_hbm, idx_hbm, out_hbm, idx_v, rows_v, sem0, sem1):
        wid = lax.axis_index("s") * info.num_cores + lax.axis_index("c")
        base = wid * cpw
        pltpu.sync_copy(idx_hbm.at[pl.ds(base, cpw)], idx_v)
        sems = (sem0, sem1)

        def start(j, buf):
            pltpu.async_copy(x_hbm.at[idx_v.at[j]], rows_v.at[buf], sems[buf])

        def finish(j, buf):
            pltpu.make_async_copy(
                x_hbm.at[idx_v.at[j]], rows_v.at[buf], sems[buf]
            ).wait()
            pltpu.sync_copy(rows_v.at[buf], out_hbm.at[pl.ds((base + j) * _CH, _CH)])

        start(0, 0)

        def body(jj, carry):
            del carry
            j = 2 * jj
            start(j + 1, 1)
            finish(j, 0)

            @pl.when(j + 2 < cpw)
            def _():
                start(j + 2, 0)

            finish(j + 1, 1)
            return 0

        lax.fori_loop(0, cpw // 2, body, 0)

    return gk


def _embed_body(a_ref, w_ref, b_ref, o_ref):
    o_ref[...] = (
        jnp.dot(a_ref[...], w_ref[...], preferred_element_type=F32) + b_ref[...]
    )


def _p1_body(g_ref, e_ref, x_ref, w1_ref, w2_ref, w3_ref, b_ref, s_ref, q_ref):
    i = pl.program_id(0)
    ab = x_ref.shape[0]
    t = jnp.dot(g_ref[...], w2_ref[...], preferred_element_type=F32)
    t += jnp.dot(e_ref[...], w3_ref[...], preferred_element_type=F32)
    s = jnp.dot(x_ref[...], w1_ref[...], preferred_element_type=F32) + b_ref[...]
    t3 = t.reshape(ab, M, 2 * AFL) + s[:, None, :]
    ps = jnp.sum(jnp.sum(t3, axis=1), axis=0, keepdims=True)
    pq = jnp.sum(jnp.sum(t3 * t3, axis=1), axis=0, keepdims=True)

    @pl.when(i == 0)
    def _():
        s_ref[...] = jnp.zeros_like(s_ref)
        q_ref[...] = jnp.zeros_like(q_ref)

    s_ref[...] += ps
    q_ref[...] += pq


def _p2_body(g_ref, e_ref, x_ref, w1f, w1c, w2f, w2c, w3f, w3c, bf, bc,
             u_ref, su_ref, qu_ref):
    i = pl.program_id(0)
    ab = x_ref.shape[0]
    g = g_ref[...]
    e = e_ref[...]
    x = x_ref[...]
    tf = jnp.dot(g, w2f[...], preferred_element_type=F32)
    tf += jnp.dot(e, w3f[...], preferred_element_type=F32)
    sf = jnp.dot(x, w1f[...], preferred_element_type=F32) + bf[...]
    tc = jnp.dot(g, w2c[...], preferred_element_type=F32)
    tc += jnp.dot(e, w3c[...], preferred_element_type=F32)
    sc = jnp.dot(x, w1c[...], preferred_element_type=F32) + bc[...]
    tf3 = tf.reshape(ab, M, AFL) + sf[:, None, :]
    tc3 = tc.reshape(ab, M, AFL) + sc[:, None, :]
    p = _sigmoid(tf3) * _softplus(tc3)
    u = jnp.sum(p, axis=1)
    u_ref[...] = u

    @pl.when(i == 0)
    def _():
        su_ref[...] = jnp.zeros_like(su_ref)
        qu_ref[...] = jnp.zeros_like(qu_ref)

    su_ref[...] += jnp.sum(u, axis=0, keepdims=True)
    qu_ref[...] += jnp.sum(u * u, axis=0, keepdims=True)


def _p3_body(x_ref, u_ref, a_ref, c_ref, o_ref):
    o_ref[...] = _softplus(x_ref[...] + u_ref[...] * a_ref[...] + c_ref[...])


def _pool_body(x_ref, u_ref, a_ref, c_ref, o_ref):
    rows = x_ref.shape[0]
    xn = _softplus(x_ref[...] + u_ref[...] * a_ref[...] + c_ref[...])
    col = lax.broadcasted_iota(jnp.int32, (64, rows), 1) // PER
    row = lax.broadcasted_iota(jnp.int32, (64, rows), 0)
    pm = jnp.where(col == row, 1.0 / PER, 0.0).astype(F32)
    o_ref[...] = jnp.dot(pm, xn, preferred_element_type=F32)[None]


def _head_body(cr_ref, gl_ref, f1a, f1b, f1bias, ow, ob, o_ref):
    h = jnp.dot(_softplus(cr_ref[...]), f1a[...], preferred_element_type=F32)
    h += jnp.dot(_softplus(gl_ref[...]), f1b[...], preferred_element_type=F32)
    h += f1bias[...]
    h2 = _softplus(h)
    o_ref[...] = jnp.sum(h2 * ow[...], axis=1, keepdims=True) + ob[...]


def _rep(shape):
    return pl.BlockSpec(shape, lambda i: (0,) * len(shape))


def kernel(atom_fea, nbr_fea, nbr_fea_idx, crystal_atom_idx, atom_type,
           nbr_type, nbr_dist, pair_type, global_fea, params):
    n, orig = atom_fea.shape
    m = nbr_fea_idx.shape[1]
    ne = n * m
    n0 = global_fea.shape[0]
    e2d = nbr_fea.reshape(ne, NBR)
    nw = 32
    ne_pad = -(-ne // (_CH * nw * 8)) * (_CH * nw * 8)
    idx_flat = nbr_fea_idx.astype(jnp.int32).reshape(ne)
    idx2d = jnp.concatenate(
        [idx_flat, jnp.zeros(ne_pad - ne, jnp.int32)]).reshape(ne_pad // _CH, _CH)

    ab_e = 2000
    x = pl.pallas_call(
        _embed_body,
        grid=(n // ab_e,),
        in_specs=[pl.BlockSpec((ab_e, orig), lambda i: (i, 0)),
                  _rep((orig, AFL)), _rep((1, AFL))],
        out_specs=pl.BlockSpec((ab_e, AFL), lambda i: (i, 0)),
        out_shape=jax.ShapeDtypeStruct((n, AFL), F32),
    )(atom_fea, params["emb_W"], params["emb_b"].reshape(1, AFL))

    gather = _make_gather(n, ne_pad, AFL)

    ab = 1000
    grid = (n // ab,)
    gspec = pl.BlockSpec((ab * M, AFL), lambda i: (i, 0))
    espec = pl.BlockSpec((ab * M, NBR), lambda i: (i, 0))
    xspec = pl.BlockSpec((ab, AFL), lambda i: (i, 0))

    nconv = len(params["convs"])
    for li, p in enumerate(params["convs"]):
        w1 = p["W"][:AFL]
        w2 = p["W"][AFL:2 * AFL]
        w3 = p["W"][2 * AFL:]
        g = gather(x, idx2d)

        ssum, qsum = pl.pallas_call(
            _p1_body,
            grid=grid,
            in_specs=[gspec, espec, xspec,
                      _rep((AFL, 2 * AFL)), _rep((AFL, 2 * AFL)),
                      _rep((NBR, 2 * AFL)), _rep((1, 2 * AFL))],
            out_specs=[_rep((1, 2 * AFL)), _rep((1, 2 * AFL))],
            out_shape=[jax.ShapeDtypeStruct((1, 2 * AFL), F32),
                       jax.ShapeDtypeStruct((1, 2 * AFL), F32)],
        )(g, e2d, x, w1, w2, w3, p["b"].reshape(1, 2 * AFL))

        mu = ssum[0] / ne
        var = qsum[0] / ne - mu * mu
        scale = p["bn1_g"] / jnp.sqrt(var + 1e-5)
        bfold = (p["b"] - mu) * scale + p["bn1_b"]
        w1s = w1 * scale[None, :]
        w2s = w2 * scale[None, :]
        w3s = w3 * scale[None, :]

        u, su, qu = pl.pallas_call(
            _p2_body,
            grid=grid,
            in_specs=[gspec, espec, xspec,
                      _rep((AFL, AFL)), _rep((AFL, AFL)),
                      _rep((AFL, AFL)), _rep((AFL, AFL)),
                      _rep((NBR, AFL)), _rep((NBR, AFL)),
                      _rep((1, AFL)), _rep((1, AFL))],
            out_specs=[pl.BlockSpec((ab, AFL), lambda i: (i, 0)),
                       _rep((1, AFL)), _rep((1, AFL))],
            out_shape=[jax.ShapeDtypeStruct((n, AFL), F32),
                       jax.ShapeDtypeStruct((1, AFL), F32),
                       jax.ShapeDtypeStruct((1, AFL), F32)],
        )(g, e2d, x,
          w1s[:, :AFL], w1s[:, AFL:], w2s[:, :AFL], w2s[:, AFL:],
          w3s[:, :AFL], w3s[:, AFL:],
          bfold[None, :AFL], bfold[None, AFL:])

        mu2 = su[0] / n
        var2 = qu[0] / n - mu2 * mu2
        a2 = p["bn2_g"] / jnp.sqrt(var2 + 1e-5)
        c2 = p["bn2_b"] - mu2 * a2

        if li + 1 < nconv:
            ab3 = 10000
            x = pl.pallas_call(
                _p3_body,
                grid=(n // ab3,),
                in_specs=[pl.BlockSpec((ab3, AFL), lambda i: (i, 0)),
                          pl.BlockSpec((ab3, AFL), lambda i: (i, 0)),
                          _rep((1, AFL)), _rep((1, AFL))],
                out_specs=pl.BlockSpec((ab3, AFL), lambda i: (i, 0)),
                out_shape=jax.ShapeDtypeStruct((n, AFL), F32),
            )(x, u, a2[None], c2[None])
        else:
            abp = 5000
            np_grid = n // abp
            pooled = pl.pallas_call(
                _pool_body,
                grid=(np_grid,),
                in_specs=[pl.BlockSpec((abp, AFL), lambda i: (i, 0)),
                          pl.BlockSpec((abp, AFL), lambda i: (i, 0)),
                          _rep((1, AFL)), _rep((1, AFL))],
                out_specs=pl.BlockSpec((1, 64, AFL), lambda i: (i, 0, 0)),
                out_shape=jax.ShapeDtypeStruct((np_grid, 64, AFL), F32),
            )(x, u, a2[None], c2[None])
            crys = pooled[:, :abp // PER, :].reshape(n0, AFL)

    gfea = global_fea.shape[1]
    hfea = params["fc1_W"].shape[1]
    out = pl.pallas_call(
        _head_body,
        in_specs=[pl.BlockSpec((n0, AFL), lambda: (0, 0)),
                  pl.BlockSpec((n0, gfea), lambda: (0, 0)),
                  pl.BlockSpec((AFL, hfea), lambda: (0, 0)),
                  pl.BlockSpec((gfea, hfea), lambda: (0, 0)),
                  pl.BlockSpec((1, hfea), lambda: (0, 0)),
                  pl.BlockSpec((1, hfea), lambda: (0, 0)),
                  pl.BlockSpec((1, 1), lambda: (0, 0))],
        out_specs=pl.BlockSpec((n0, 1), lambda: (0, 0)),
        out_shape=jax.ShapeDtypeStruct((n0, 1), F32),
    )(crys, global_fea, params["fc1_W"][:AFL], params["fc1_W"][AFL:],
      params["fc1_b"].reshape(1, hfea), params["out_W"].reshape(1, hfea),
      params["out_b"].reshape(1, 1))
    return out


# speedup vs baseline: 1.6924x; 1.1316x over previous
import functools

import jax
import jax.numpy as jnp
from jax import lax
from jax.experimental import pallas as pl
from jax.experimental.pallas import tpu as pltpu
from jax.experimental.pallas import tpu_sc as plsc

F32 = jnp.float32
AFL = 64
M = 16
NBR = 16
PER = 100

_CH = 128


def _softplus(x):
    return jnp.maximum(x, 0.0) + jnp.log(1.0 + jnp.exp(-jnp.abs(x)))


def _sigmoid(x):
    z = jnp.exp(-jnp.abs(x))
    return jnp.where(x >= 0, 1.0 / (1.0 + z), z / (1.0 + z))


_KG = 4


def _make_gather(n_rows, n_edges, feat, dtype):
    info = plsc.get_sparse_core_info()
    nw = info.num_cores * info.num_subcores
    n_chunks = n_edges // _CH
    assert n_chunks * _CH == n_edges and n_chunks % nw == 0
    cpw = n_chunks // nw
    ng = cpw // _KG
    grows = _KG * _CH
    assert ng * _KG == cpw and ng >= 4
    mesh = plsc.VectorSubcoreMesh(core_axis_name="c", subcore_axis_name="s")

    @functools.partial(
        pl.kernel,
        mesh=mesh,
        compiler_params=pltpu.CompilerParams(use_tc_tiling_on_sc=False),
        out_type=jax.ShapeDtypeStruct((n_edges, feat), dtype),
        scratch_types=[
            pltpu.VMEM((cpw, _CH), jnp.int32),
            pltpu.VMEM((3, grows, feat), dtype),
            pltpu.SemaphoreType.DMA,
            pltpu.SemaphoreType.DMA,
            pltpu.SemaphoreType.DMA,
            pltpu.SemaphoreType.DMA,
            pltpu.SemaphoreType.DMA,
            pltpu.SemaphoreType.DMA,
        ],
    )
    def gk(x_hbm, idx_hbm, out_hbm, idx_v, rows_v,
           sg0, sg1, sg2, ss0, ss1, ss2):
        wid = lax.axis_index("s") * info.num_cores + lax.axis_index("c")
        base = wid * cpw
        pltpu.sync_copy(idx_hbm.at[pl.ds(base, cpw)], idx_v)
        sg = (sg0, sg1, sg2)
        ss = (ss0, ss1, ss2)

        def fire(g, b):
            for k in range(_KG):
                pltpu.async_copy(
                    x_hbm.at[idx_v.at[g * _KG + k]],
                    rows_v.at[b].at[pl.ds(k * _CH, _CH)], sg[b])

        def drain_gather(g, b):
            for k in range(_KG):
                pltpu.make_async_copy(
                    x_hbm.at[idx_v.at[g * _KG + k]],
                    rows_v.at[b].at[pl.ds(k * _CH, _CH)], sg[b]).wait()

        def out_slice(g):
            return out_hbm.at[pl.ds((base + g * _KG) * _CH, grows)]

        def start_store(g, b):
            pltpu.async_copy(rows_v.at[b], out_slice(g), ss[b])

        def wait_store(g, b):
            pltpu.make_async_copy(rows_v.at[b], out_slice(g), ss[b]).wait()

        fire(0, 0)
        fire(1, 1)

        def step(g, b, bprev):
            drain_gather(g, b)
            start_store(g, b)

            @pl.when(g + 2 < ng)
            def _():
                @pl.when(g >= 1)
                def _():
                    wait_store(g - 1, bprev)
                fire(g + 2, bprev)

        def body(gg, carry):
            del carry
            g = 3 * gg
            step(g, 0, 2)
            step(g + 1, 1, 0)
            step(g + 2, 2, 1)
            return 0

        nloop = ng // 3
        lax.fori_loop(0, nloop, body, 0)
        for gtail in range(nloop * 3, ng):
            drain_gather(gtail, gtail % 3)
            start_store(gtail, gtail % 3)
        for g in range(ng - 3, ng):
            wait_store(g, g % 3)

    return gk


def _embed_body(a_ref, w_ref, b_ref, o_ref):
    o_ref[...] = (
        jnp.dot(a_ref[...], w_ref[...], preferred_element_type=F32) + b_ref[...]
    )


def _p1_body(g_ref, e_ref, x_ref, w1_ref, w2_ref, w3_ref, b_ref, s_ref, q_ref):
    i = pl.program_id(0)
    ab = x_ref.shape[0]
    t = jnp.dot(g_ref[...], w2_ref[...], preferred_element_type=F32)
    t += jnp.dot(e_ref[...], w3_ref[...], preferred_element_type=F32)
    s = jnp.dot(x_ref[...], w1_ref[...], preferred_element_type=F32) + b_ref[...]
    t3 = t.reshape(ab, M, 2 * AFL) + s[:, None, :]
    ps = jnp.sum(jnp.sum(t3, axis=1), axis=0, keepdims=True)
    pq = jnp.sum(jnp.sum(t3 * t3, axis=1), axis=0, keepdims=True)

    @pl.when(i == 0)
    def _():
        s_ref[...] = jnp.zeros_like(s_ref)
        q_ref[...] = jnp.zeros_like(q_ref)

    s_ref[...] += ps
    q_ref[...] += pq


def _p2_body(g_ref, e_ref, x_ref, w1f, w1c, w2f, w2c, w3f, w3c, bf, bc,
             af, cf, ac, cc, u_ref, su_ref, qu_ref):
    i = pl.program_id(0)
    ab = x_ref.shape[0]
    g = g_ref[...]
    e = e_ref[...]
    x = x_ref[...]
    tf = jnp.dot(g, w2f[...], preferred_element_type=F32)
    tf += jnp.dot(e, w3f[...], preferred_element_type=F32)
    sf = jnp.dot(x, w1f[...], preferred_element_type=F32) + bf[...]
    tc = jnp.dot(g, w2c[...], preferred_element_type=F32)
    tc += jnp.dot(e, w3c[...], preferred_element_type=F32)
    sc = jnp.dot(x, w1c[...], preferred_element_type=F32) + bc[...]
    tf3 = (tf.reshape(ab, M, AFL) + sf[:, None, :]) * af[...] + cf[...]
    tc3 = (tc.reshape(ab, M, AFL) + sc[:, None, :]) * ac[...] + cc[...]
    p = _sigmoid(tf3) * _softplus(tc3)
    u = jnp.sum(p, axis=1)
    u_ref[...] = u

    @pl.when(i == 0)
    def _():
        su_ref[...] = jnp.zeros_like(su_ref)
        qu_ref[...] = jnp.zeros_like(qu_ref)

    su_ref[...] += jnp.sum(u, axis=0, keepdims=True)
    qu_ref[...] += jnp.sum(u * u, axis=0, keepdims=True)


def _p3_body(x_ref, u_ref, a_ref, c_ref, o_ref):
    o_ref[...] = _softplus(x_ref[...] + u_ref[...] * a_ref[...] + c_ref[...])


def _pool_body(x_ref, u_ref, a_ref, c_ref, o_ref):
    rows = x_ref.shape[0]
    xn = _softplus(x_ref[...] + u_ref[...] * a_ref[...] + c_ref[...])
    col = lax.broadcasted_iota(jnp.int32, (64, rows), 1) // PER
    row = lax.broadcasted_iota(jnp.int32, (64, rows), 0)
    pm = jnp.where(col == row, 1.0 / PER, 0.0).astype(F32)
    o_ref[...] = jnp.dot(pm, xn, preferred_element_type=F32)[None]


def _head_body(cr_ref, gl_ref, f1a, f1b, f1bias, ow, ob, o_ref):
    h = jnp.dot(_softplus(cr_ref[...]), f1a[...], preferred_element_type=F32)
    h += jnp.dot(_softplus(gl_ref[...]), f1b[...], preferred_element_type=F32)
    h += f1bias[...]
    h2 = _softplus(h)
    o_ref[...] = jnp.sum(h2 * ow[...], axis=1, keepdims=True) + ob[...]


def _rep(shape):
    return pl.BlockSpec(shape, lambda i: (0,) * len(shape))


def kernel(atom_fea, nbr_fea, nbr_fea_idx, crystal_atom_idx, atom_type,
           nbr_type, nbr_dist, pair_type, global_fea, params):
    n, orig = atom_fea.shape
    m = nbr_fea_idx.shape[1]
    ne = n * m
    n0 = global_fea.shape[0]
    e2d = nbr_fea.reshape(ne, NBR)
    nw = 32
    ne_pad = -(-ne // (_CH * nw * 8)) * (_CH * nw * 8)
    idx_flat = nbr_fea_idx.astype(jnp.int32).reshape(ne)
    idx2d = jnp.concatenate(
        [idx_flat, jnp.zeros(ne_pad - ne, jnp.int32)]).reshape(ne_pad // _CH, _CH)

    ab_e = 2000
    x = pl.pallas_call(
        _embed_body,
        grid=(n // ab_e,),
        in_specs=[pl.BlockSpec((ab_e, orig), lambda i: (i, 0)),
                  _rep((orig, AFL)), _rep((1, AFL))],
        out_specs=pl.BlockSpec((ab_e, AFL), lambda i: (i, 0)),
        out_shape=jax.ShapeDtypeStruct((n, AFL), F32),
    )(atom_fea, params["emb_W"], params["emb_b"].reshape(1, AFL))

    gather = _make_gather(n, ne_pad, AFL, jnp.bfloat16)
    e2dh = e2d.astype(jnp.bfloat16)

    ab = 1000
    grid = (n // ab,)
    gspec = pl.BlockSpec((ab * M, AFL), lambda i: (i, 0))
    espec = pl.BlockSpec((ab * M, NBR), lambda i: (i, 0))
    xspec = pl.BlockSpec((ab, AFL), lambda i: (i, 0))

    nconv = len(params["convs"])
    for li, p in enumerate(params["convs"]):
        w1h = p["W"][:AFL].astype(jnp.bfloat16)
        w2h = p["W"][AFL:2 * AFL].astype(jnp.bfloat16)
        w3h = p["W"][2 * AFL:].astype(jnp.bfloat16)
        xh = x.astype(jnp.bfloat16)
        g = gather(xh, idx2d)

        ssum, qsum = pl.pallas_call(
            _p1_body,
            grid=grid,
            in_specs=[gspec, espec, xspec,
                      _rep((AFL, 2 * AFL)), _rep((AFL, 2 * AFL)),
                      _rep((NBR, 2 * AFL)), _rep((1, 2 * AFL))],
            out_specs=[_rep((1, 2 * AFL)), _rep((1, 2 * AFL))],
            out_shape=[jax.ShapeDtypeStruct((1, 2 * AFL), F32),
                       jax.ShapeDtypeStruct((1, 2 * AFL), F32)],
        )(g, e2dh, xh, w1h, w2h, w3h, p["b"].reshape(1, 2 * AFL))

        mu = ssum[0] / ne
        var = qsum[0] / ne - mu * mu
        a1 = p["bn1_g"] / jnp.sqrt(var + 1e-5)
        c1 = p["bn1_b"] - mu * a1

        u, su, qu = pl.pallas_call(
            _p2_body,
            grid=grid,
            in_specs=[gspec, espec, xspec,
                      _rep((AFL, AFL)), _rep((AFL, AFL)),
                      _rep((AFL, AFL)), _rep((AFL, AFL)),
                      _rep((NBR, AFL)), _rep((NBR, AFL)),
                      _rep((1, AFL)), _rep((1, AFL)),
                      _rep((1, AFL)), _rep((1, AFL)),
                      _rep((1, AFL)), _rep((1, AFL))],
            out_specs=[pl.BlockSpec((ab, AFL), lambda i: (i, 0)),
                       _rep((1, AFL)), _rep((1, AFL))],
            out_shape=[jax.ShapeDtypeStruct((n, AFL), F32),
                       jax.ShapeDtypeStruct((1, AFL), F32),
                       jax.ShapeDtypeStruct((1, AFL), F32)],
        )(g, e2dh, xh,
          w1h[:, :AFL], w1h[:, AFL:], w2h[:, :AFL], w2h[:, AFL:],
          w3h[:, :AFL], w3h[:, AFL:],
          p["b"][None, :AFL], p["b"][None, AFL:],
          a1[None, :AFL], c1[None, :AFL], a1[None, AFL:], c1[None, AFL:])

        mu2 = su[0] / n
        var2 = qu[0] / n - mu2 * mu2
        a2 = p["bn2_g"] / jnp.sqrt(var2 + 1e-5)
        c2 = p["bn2_b"] - mu2 * a2

        if li + 1 < nconv:
            ab3 = 10000
            x = pl.pallas_call(
                _p3_body,
                grid=(n // ab3,),
                in_specs=[pl.BlockSpec((ab3, AFL), lambda i: (i, 0)),
                          pl.BlockSpec((ab3, AFL), lambda i: (i, 0)),
                          _rep((1, AFL)), _rep((1, AFL))],
                out_specs=pl.BlockSpec((ab3, AFL), lambda i: (i, 0)),
                out_shape=jax.ShapeDtypeStruct((n, AFL), F32),
            )(x, u, a2[None], c2[None])
        else:
            abp = 5000
            np_grid = n // abp
            pooled = pl.pallas_call(
                _pool_body,
                grid=(np_grid,),
                in_specs=[pl.BlockSpec((abp, AFL), lambda i: (i, 0)),
                          pl.BlockSpec((abp, AFL), lambda i: (i, 0)),
                          _rep((1, AFL)), _rep((1, AFL))],
                out_specs=pl.BlockSpec((1, 64, AFL), lambda i: (i, 0, 0)),
                out_shape=jax.ShapeDtypeStruct((np_grid, 64, AFL), F32),
            )(x, u, a2[None], c2[None])
            crys = pooled[:, :abp // PER, :].reshape(n0, AFL)

    gfea = global_fea.shape[1]
    hfea = params["fc1_W"].shape[1]
    out = pl.pallas_call(
        _head_body,
        in_specs=[pl.BlockSpec((n0, AFL), lambda: (0, 0)),
                  pl.BlockSpec((n0, gfea), lambda: (0, 0)),
                  pl.BlockSpec((AFL, hfea), lambda: (0, 0)),
                  pl.BlockSpec((gfea, hfea), lambda: (0, 0)),
                  pl.BlockSpec((1, hfea), lambda: (0, 0)),
                  pl.BlockSpec((1, hfea), lambda: (0, 0)),
                  pl.BlockSpec((1, 1), lambda: (0, 0))],
        out_specs=pl.BlockSpec((n0, 1), lambda: (0, 0)),
        out_shape=jax.ShapeDtypeStruct((n0, 1), F32),
    )(crys, global_fea, params["fc1_W"][:AFL], params["fc1_W"][AFL:],
      params["fc1_b"].reshape(1, hfea), params["out_W"].reshape(1, hfea),
      params["out_b"].reshape(1, 1))
    return out
